# baseline (speedup 1.0000x reference)
import jax
import jax.numpy as jnp
from jax.experimental import pallas as pl

N, U, V = 16, 512, 1024
P = 0.5
UB = 256


def _body(y_ref, pf_ref, lp_ref, loss_ref, ilm_ref, kl_ref):
    y = y_ref[0, 0, :]
    lp = lp_ref[0]
    pf = pf_ref[0]
    ycol = y[:, None]
    iota = jax.lax.broadcasted_iota(jnp.int32, (UB, V), 1)
    hit = (iota == ycol) & (ycol != 0)
    eye = jnp.where(hit, jnp.float32(1.0 - P), jnp.float32(0.0))
    ilm = eye * lp
    kl = jnp.float32(P) * (pf * lp)
    ilm_ref[0] = ilm
    kl_ref[0] = kl
    loss_ref[0] = ilm + kl


def kernel(p_fixed, logp_train, y_padded):
    y3 = y_padded.reshape(N, 1, U)
    grid = (N, U // UB)
    out_shape = jax.ShapeDtypeStruct((N, U, V), jnp.float32)
    return pl.pallas_call(
        _body,
        grid=grid,
        in_specs=[
            pl.BlockSpec((1, 1, UB), lambda n, u: (n, 0, u)),
            pl.BlockSpec((1, UB, V), lambda n, u: (n, u, 0)),
            pl.BlockSpec((1, UB, V), lambda n, u: (n, u, 0)),
        ],
        out_specs=[
            pl.BlockSpec((1, UB, V), lambda n, u: (n, u, 0)),
            pl.BlockSpec((1, UB, V), lambda n, u: (n, u, 0)),
            pl.BlockSpec((1, UB, V), lambda n, u: (n, u, 0)),
        ],
        out_shape=[out_shape, out_shape, out_shape],
    )(y3, p_fixed, logp_train)


# speedup vs baseline: 1.0524x; 1.0524x over previous
import jax
import jax.numpy as jnp
from jax.experimental import pallas as pl

N, U, V = 16, 512, 1024
P = 0.5
UB = 512


def _body(y_ref, pf_ref, lp_ref, loss_ref, ilm_ref, kl_ref):
    y = y_ref[0, 0, :]
    lp = lp_ref[0]
    pf = pf_ref[0]
    ycol = y[:, None]
    iota = jax.lax.broadcasted_iota(jnp.int32, (UB, V), 1)
    hit = (iota == ycol) & (ycol != 0)
    eye = jnp.where(hit, jnp.float32(1.0 - P), jnp.float32(0.0))
    ilm = eye * lp
    kl = jnp.float32(P) * (pf * lp)
    ilm_ref[0] = ilm
    kl_ref[0] = kl
    loss_ref[0] = ilm + kl


def kernel(p_fixed, logp_train, y_padded):
    y3 = y_padded.reshape(N, 1, U)
    grid = (N, U // UB)
    out_shape = jax.ShapeDtypeStruct((N, U, V), jnp.float32)
    return pl.pallas_call(
        _body,
        grid=grid,
        in_specs=[
            pl.BlockSpec((1, 1, UB), lambda n, u: (n, 0, u)),
            pl.BlockSpec((1, UB, V), lambda n, u: (n, u, 0)),
            pl.BlockSpec((1, UB, V), lambda n, u: (n, u, 0)),
        ],
        out_specs=[
            pl.BlockSpec((1, UB, V), lambda n, u: (n, u, 0)),
            pl.BlockSpec((1, UB, V), lambda n, u: (n, u, 0)),
            pl.BlockSpec((1, UB, V), lambda n, u: (n, u, 0)),
        ],
        out_shape=[out_shape, out_shape, out_shape],
    )(y3, p_fixed, logp_train)


# speedup vs baseline: 1.0999x; 1.0451x over previous
import jax
import jax.numpy as jnp
from jax.experimental import pallas as pl

N, U, V = 16, 512, 1024
P = 0.5
UB = 512


NB = 2


def _body(y_ref, pf_ref, lp_ref, loss_ref, ilm_ref, kl_ref):
    y = y_ref[:, 0, :]
    lp = lp_ref[...]
    pf = pf_ref[...]
    ycol = y[:, :, None]
    iota = jax.lax.broadcasted_iota(jnp.int32, (NB, U, V), 2)
    hit = (iota == ycol) & (ycol != 0)
    eye = jnp.where(hit, jnp.float32(1.0 - P), jnp.float32(0.0))
    ilm = eye * lp
    kl = jnp.float32(P) * (pf * lp)
    ilm_ref[...] = ilm
    kl_ref[...] = kl
    loss_ref[...] = ilm + kl


def kernel(p_fixed, logp_train, y_padded):
    y3 = y_padded.reshape(N, 1, U)
    grid = (N // NB,)
    out_shape = jax.ShapeDtypeStruct((N, U, V), jnp.float32)
    return pl.pallas_call(
        _body,
        grid=grid,
        in_specs=[
            pl.BlockSpec((NB, 1, U), lambda n: (n, 0, 0)),
            pl.BlockSpec((NB, U, V), lambda n: (n, 0, 0)),
            pl.BlockSpec((NB, U, V), lambda n: (n, 0, 0)),
        ],
        out_specs=[
            pl.BlockSpec((NB, U, V), lambda n: (n, 0, 0)),
            pl.BlockSpec((NB, U, V), lambda n: (n, 0, 0)),
            pl.BlockSpec((NB, U, V), lambda n: (n, 0, 0)),
        ],
        out_shape=[out_shape, out_shape, out_shape],
    )(y3, p_fixed, logp_train)
